# baseline (speedup 1.0000x reference)
import functools

import jax
import jax.numpy as jnp
from jax import lax
from jax.experimental import pallas as pl
from jax.experimental.pallas import tpu as pltpu
from jax.experimental.pallas import tpu_sc as plsc

_B, _S = 1024, 200
_L = 4
_VL, _VT = 100000, 512
_D = 32
_NW = 32
_BT = _B // 128
_NCH = _S * _BT
_NCHW = _NCH // _NW


def _make_kernel():
    mesh = plsc.VectorSubcoreMesh(core_axis_name="c", subcore_axis_name="s")

    @functools.partial(
        pl.kernel,
        mesh=mesh,
        out_type=jax.ShapeDtypeStruct((_S, 8, _BT, 8, 128), jnp.float32),
        compiler_params=pltpu.CompilerParams(
            use_tc_tiling_on_sc=False, needs_layout_passes=False),
        scratch_types=[
            pltpu.VMEM((2, _L, 128), jnp.int32),
            pltpu.VMEM((2, _L, 128), jnp.int32),
            pltpu.VMEM((2, _L, 128, _D), jnp.float32),
            pltpu.VMEM((2, _L, 128, _D), jnp.float32),
            pltpu.VMEM((2, 8, 8, 128), jnp.float32),
            pltpu.VMEM((2 * _L, 16), jnp.float32),
            pltpu.SemaphoreType.DMA,
            pltpu.SemaphoreType.DMA,
            pltpu.SemaphoreType.DMA,
            pltpu.SemaphoreType.DMA,
            pltpu.SemaphoreType.DMA,
            pltpu.SemaphoreType.DMA,
        ],
    )
    def k(x_hbm, t_hbm, loc_hbm, time_hbm, w_hbm, out_hbm,
          idx_x, idx_t, rows_x, rows_t, out_v, w_v,
          sem_i0, sem_i1, sem_g0, sem_g1, sem_o0, sem_o1):
        wid = lax.axis_index("s") * 2 + lax.axis_index("c")
        g0 = wid * _NCHW
        sem_i = [sem_i0, sem_i1]
        sem_g = [sem_g0, sem_g1]
        sem_o = [sem_o0, sem_o1]

        pltpu.sync_copy(w_hbm, w_v)
        ws = [w_v[j] for j in range(2 * _L)]
        lane = lax.iota(jnp.int32, 16)
        cvecs = [jnp.full((16,), d, jnp.int32) for d in range(_D)]

        def chunk_slices(g):
            return g // _BT, g % _BT

        def stage_idx(par, g):
            s, bc = chunk_slices(g)
            pltpu.async_copy(x_hbm.at[s, bc], idx_x.at[par], sem_i[par])
            pltpu.async_copy(t_hbm.at[s, bc], idx_t.at[par], sem_i[par])

        def wait_idx(par):
            pltpu.make_async_copy(
                x_hbm.at[0, 0], idx_x.at[par], sem_i[par]).wait()
            pltpu.make_async_copy(
                t_hbm.at[0, 0], idx_t.at[par], sem_i[par]).wait()

        def issue_gathers(par):
            for l in range(_L):
                pltpu.async_copy(
                    loc_hbm.at[l].at[idx_x.at[par, l]],
                    rows_x.at[par, l], sem_g[par])
                pltpu.async_copy(
                    time_hbm.at[l].at[idx_t.at[par, l]],
                    rows_t.at[par, l], sem_g[par])

        def wait_gathers(par):
            for l in range(_L):
                pltpu.make_async_copy(
                    loc_hbm.at[0, pl.ds(0, 128), :], rows_x.at[par, l],
                    sem_g[par]).wait()
                pltpu.make_async_copy(
                    loc_hbm.at[0, pl.ds(0, 128), :], rows_t.at[par, l],
                    sem_g[par]).wait()

        def compute(par):
            @plsc.parallel_loop(0, 8)
            def _(grp):
                rvec = grp * 16 + lane
                for d in range(_D):
                    a = ws[0] * plsc.load_gather(
                        rows_x.at[par, 0], [rvec, cvecs[d]])
                    for l in range(1, _L):
                        a = a + ws[l] * plsc.load_gather(
                            rows_x.at[par, l], [rvec, cvecs[d]])
                    out_v[par, d >> 3, d & 7, pl.ds(grp * 16, 16)] = a
                    b = ws[_L] * plsc.load_gather(
                        rows_t.at[par, 0], [rvec, cvecs[d]])
                    for l in range(1, _L):
                        b = b + ws[_L + l] * plsc.load_gather(
                            rows_t.at[par, l], [rvec, cvecs[d]])
                    dd = _D + d
                    out_v[par, dd >> 3, dd & 7, pl.ds(grp * 16, 16)] = b

        def issue_out(par, g):
            s, bc = chunk_slices(g)
            for dt in range(8):
                pltpu.async_copy(out_v.at[par, dt],
                                 out_hbm.at[s, dt, bc], sem_o[par])

        def drain_out(par, g):
            s, bc = chunk_slices(g)
            for dt in range(8):
                pltpu.make_async_copy(
                    out_v.at[par, dt], out_hbm.at[s, dt, bc],
                    sem_o[par]).wait()

        stage_idx(0, g0)
        stage_idx(1, g0 + 1)
        wait_idx(0)
        issue_gathers(0)

        def super_body(i, carry):
            for par in range(2):
                g = g0 + 2 * i + par
                nxt = 1 - par
                wait_gathers(par)

                @pl.when(i < _NCHW // 2 - 1)
                def _prefetch_idx():
                    stage_idx(par, g + 2)

                def _launch_next():
                    wait_idx(nxt)
                    issue_gathers(nxt)

                if par == 0:
                    _launch_next()
                else:
                    pl.when(i < _NCHW // 2 - 1)(_launch_next)

                @pl.when(i > 0)
                def _drain_out():
                    drain_out(par, g - 2)

                compute(par)
                issue_out(par, g)
            return carry

        lax.fori_loop(0, _NCHW // 2, super_body, 0)

        last = g0 + _NCHW - 2
        drain_out(0, last)
        drain_out(1, last + 1)

    return k


_k = _make_kernel()


def kernel(x, t, loc_tables, time_tables, x_weights, t_weights):
    xp = x.astype(jnp.int32).transpose(1, 2, 0).reshape(_S, _L, _BT, 128)
    xp = xp.transpose(0, 2, 1, 3)
    tp = t.astype(jnp.int32).transpose(1, 2, 0).reshape(_S, _L, _BT, 128)
    tp = tp.transpose(0, 2, 1, 3)
    w_all = jnp.broadcast_to(
        jnp.concatenate([x_weights, t_weights])[:, None], (2 * _L, 16))
    op = _k(xp, tp, loc_tables, time_tables, w_all)
    out = op.transpose(2, 4, 0, 1, 3).reshape(_B, _S, 2 * _D)
    return out


# speedup vs baseline: 2.9287x; 2.9287x over previous
import functools

import jax
import jax.numpy as jnp
from jax import lax
from jax.experimental import pallas as pl
from jax.experimental.pallas import tpu as pltpu
from jax.experimental.pallas import tpu_sc as plsc

_B, _S = 1024, 200
_L = 4
_VL, _VT = 100000, 512
_D = 32
_NW = 32
_BT = _B // 128
_NCH = _S * _BT
_NCHW = _NCH // _NW
_P = 129


def _make_kernel():
    mesh = plsc.VectorSubcoreMesh(core_axis_name="c", subcore_axis_name="s")

    @functools.partial(
        pl.kernel,
        mesh=mesh,
        out_type=jax.ShapeDtypeStruct((_S, 8, _BT, 8, 128), jnp.float32),
        compiler_params=pltpu.CompilerParams(
            use_tc_tiling_on_sc=False, needs_layout_passes=False),
        scratch_types=[
            pltpu.VMEM((2, _L, 128), jnp.int32),
            pltpu.VMEM((2, _L, 128), jnp.int32),
            pltpu.VMEM((2, _L, 128, _D), jnp.float32),
            pltpu.VMEM((2, _L, 128, _D), jnp.float32),
            pltpu.VMEM((2, 8, 8, _P), jnp.float32),
            pltpu.VMEM((2 * _L, 16), jnp.float32),
            pltpu.SemaphoreType.DMA,
            pltpu.SemaphoreType.DMA,
            pltpu.SemaphoreType.DMA,
            pltpu.SemaphoreType.DMA,
            pltpu.SemaphoreType.DMA,
            pltpu.SemaphoreType.DMA,
        ],
    )
    def k(x_hbm, t_hbm, loc_hbm, time_hbm, w_hbm, out_hbm,
          idx_x, idx_t, rows_x, rows_t, out_v, w_v,
          sem_i0, sem_i1, sem_g0, sem_g1, sem_o0, sem_o1):
        wid = lax.axis_index("s") * 2 + lax.axis_index("c")
        g0 = wid * _NCHW
        sem_i = [sem_i0, sem_i1]
        sem_g = [sem_g0, sem_g1]
        sem_o = [sem_o0, sem_o1]

        pltpu.sync_copy(w_hbm, w_v)
        ws = [w_v[j] for j in range(2 * _L)]
        lane = lax.iota(jnp.int32, 16)
        dts = [(jnp.int32(q * 16) + lane) >> 3 for q in range(4)]
        drs = [(jnp.int32(q * 16) + lane) & 7 for q in range(4)]

        def chunk_slices(g):
            return g // _BT, g % _BT

        def stage_idx(par, g):
            s, bc = chunk_slices(g)
            pltpu.async_copy(x_hbm.at[s, bc], idx_x.at[par], sem_i[par])
            pltpu.async_copy(t_hbm.at[s, bc], idx_t.at[par], sem_i[par])

        def wait_idx(par):
            pltpu.make_async_copy(
                x_hbm.at[0, 0], idx_x.at[par], sem_i[par]).wait()
            pltpu.make_async_copy(
                t_hbm.at[0, 0], idx_t.at[par], sem_i[par]).wait()

        def add_offsets(par):
            for l in range(1, _L):
                off = jnp.int32(l * _VL)
                for kk in range(8):
                    sl = pl.ds(kk * 16, 16)
                    idx_x[par, l, sl] = idx_x[par, l, sl] + off

        def issue_gathers(par):
            for l in range(_L):
                pltpu.async_copy(
                    loc_hbm.at[idx_x.at[par, l]],
                    rows_x.at[par, l], sem_g[par])
                pltpu.async_copy(
                    time_hbm.at[l].at[idx_t.at[par, l]],
                    rows_t.at[par, l], sem_g[par])

        def wait_gathers(par):
            for l in range(_L):
                pltpu.make_async_copy(
                    loc_hbm.at[pl.ds(0, 128), :], rows_x.at[par, l],
                    sem_g[par]).wait()
                pltpu.make_async_copy(
                    loc_hbm.at[pl.ds(0, 128), :], rows_t.at[par, l],
                    sem_g[par]).wait()

        def compute(par):
            @plsc.parallel_loop(0, 128, unroll=2)
            def _(tok):
                tk = jnp.full((16,), tok, jnp.int32)
                vs = []
                for p in range(2):
                    sl = pl.ds(p * 16, 16)
                    a = ws[0] * rows_x[par, 0, tok, sl]
                    for l in range(1, _L):
                        a = a + ws[l] * rows_x[par, l, tok, sl]
                    vs.append(a)
                for p in range(2):
                    sl = pl.ds(p * 16, 16)
                    b = ws[_L] * rows_t[par, 0, tok, sl]
                    for l in range(1, _L):
                        b = b + ws[_L + l] * rows_t[par, l, tok, sl]
                    vs.append(b)
                for q in range(4):
                    plsc.store_scatter(out_v.at[par], [dts[q], drs[q], tk],
                                       vs[q])

        def issue_out(par, g):
            s, bc = chunk_slices(g)
            for dt in range(8):
                pltpu.async_copy(out_v.at[par, dt, :, pl.ds(0, 128)],
                                 out_hbm.at[s, dt, bc], sem_o[par])

        def drain_out(par, g):
            s, bc = chunk_slices(g)
            for dt in range(8):
                pltpu.make_async_copy(
                    out_v.at[par, dt, :, pl.ds(0, 128)],
                    out_hbm.at[s, dt, bc], sem_o[par]).wait()

        stage_idx(0, g0)
        stage_idx(1, g0 + 1)
        wait_idx(0)
        add_offsets(0)
        issue_gathers(0)

        def super_body(i, carry):
            for par in range(2):
                g = g0 + 2 * i + par
                nxt = 1 - par
                wait_gathers(par)

                @pl.when(i < _NCHW // 2 - 1)
                def _prefetch_idx():
                    stage_idx(par, g + 2)

                def _launch_next():
                    wait_idx(nxt)
                    add_offsets(nxt)
                    issue_gathers(nxt)

                if par == 0:
                    _launch_next()
                else:
                    pl.when(i < _NCHW // 2 - 1)(_launch_next)

                @pl.when(i > 0)
                def _drain_out():
                    drain_out(par, g - 2)

                compute(par)
                issue_out(par, g)
            return carry

        lax.fori_loop(0, _NCHW // 2, super_body, 0)

        last = g0 + _NCHW - 2
        drain_out(0, last)
        drain_out(1, last + 1)

    return k


_k = _make_kernel()


def kernel(x, t, loc_tables, time_tables, x_weights, t_weights):
    xp = x.astype(jnp.int32).transpose(1, 2, 0).reshape(_S, _L, _BT, 128)
    xp = xp.transpose(0, 2, 1, 3)
    tp = t.astype(jnp.int32).transpose(1, 2, 0).reshape(_S, _L, _BT, 128)
    tp = tp.transpose(0, 2, 1, 3)
    lt = lax.optimization_barrier(loc_tables.reshape(_VL, _L * _D))
    loc_flat = lt.reshape(_L * _VL, _D)
    w_all = jnp.broadcast_to(
        jnp.concatenate([x_weights, t_weights])[:, None], (2 * _L, 16))
    op = _k(xp, tp, loc_flat, time_tables, w_all)
    out = op.transpose(2, 4, 0, 1, 3).reshape(_B, _S, 2 * _D)
    return out


# speedup vs baseline: 2.9894x; 1.0207x over previous
import functools

import jax
import jax.numpy as jnp
from jax import lax
from jax.experimental import pallas as pl
from jax.experimental.pallas import tpu as pltpu
from jax.experimental.pallas import tpu_sc as plsc

_B, _S = 1024, 200
_L = 4
_VL, _VT = 100000, 512
_D = 32
_NW = 32
_BT = _B // 128
_NCH = _S * _BT
_NCHW = _NCH // _NW
_P = 129


def _make_kernel():
    mesh = plsc.VectorSubcoreMesh(core_axis_name="c", subcore_axis_name="s")

    @functools.partial(
        pl.kernel,
        mesh=mesh,
        out_type=jax.ShapeDtypeStruct((_S, 8, _BT, 8, 128), jnp.float32),
        compiler_params=pltpu.CompilerParams(
            use_tc_tiling_on_sc=False, needs_layout_passes=False),
        scratch_types=[
            pltpu.VMEM((2, _L, 128), jnp.int32),
            pltpu.VMEM((2, _L, 128), jnp.int32),
            pltpu.VMEM((2, _L, 128, _D), jnp.float32),
            pltpu.VMEM((2, _L, 128, _D), jnp.float32),
            pltpu.VMEM((2, 8, 8, _P), jnp.float32),
            pltpu.VMEM((2 * _L, 16), jnp.float32),
            pltpu.SemaphoreType.DMA,
            pltpu.SemaphoreType.DMA,
            pltpu.SemaphoreType.DMA,
            pltpu.SemaphoreType.DMA,
            pltpu.SemaphoreType.DMA,
            pltpu.SemaphoreType.DMA,
        ],
    )
    def k(x_hbm, t_hbm, loc_hbm, time_hbm, w_hbm, out_hbm,
          idx_x, idx_t, rows_x, rows_t, out_v, w_v,
          sem_i0, sem_i1, sem_g0, sem_g1, sem_o0, sem_o1):
        wid = lax.axis_index("s") * 2 + lax.axis_index("c")
        g0 = wid * _NCHW
        sem_i = [sem_i0, sem_i1]
        sem_g = [sem_g0, sem_g1]
        sem_o = [sem_o0, sem_o1]

        pltpu.sync_copy(w_hbm, w_v)
        ws = [w_v[j] for j in range(2 * _L)]
        lane = lax.iota(jnp.int32, 16)
        dts = [(jnp.int32(q * 16) + lane) >> 3 for q in range(4)]
        drs = [(jnp.int32(q * 16) + lane) & 7 for q in range(4)]

        def chunk_slices(g):
            return g // _BT, g % _BT

        def stage_idx(par, g):
            s, bc = chunk_slices(g)
            pltpu.async_copy(x_hbm.at[s, bc], idx_x.at[par], sem_i[par])
            pltpu.async_copy(t_hbm.at[s, bc], idx_t.at[par], sem_i[par])

        def wait_idx(par):
            pltpu.make_async_copy(
                x_hbm.at[0, 0], idx_x.at[par], sem_i[par]).wait()
            pltpu.make_async_copy(
                t_hbm.at[0, 0], idx_t.at[par], sem_i[par]).wait()

        def add_offsets(par):
            for l in range(1, _L):
                off = jnp.int32(l * _VL)
                for kk in range(8):
                    sl = pl.ds(kk * 16, 16)
                    idx_x[par, l, sl] = idx_x[par, l, sl] + off

        def issue_gathers(par):
            for l in range(_L):
                pltpu.async_copy(
                    loc_hbm.at[idx_x.at[par, l]],
                    rows_x.at[par, l], sem_g[par])
                pltpu.async_copy(
                    time_hbm.at[l].at[idx_t.at[par, l]],
                    rows_t.at[par, l], sem_g[par])

        def wait_gathers(par):
            for l in range(_L):
                pltpu.make_async_copy(
                    loc_hbm.at[pl.ds(0, 128), :], rows_x.at[par, l],
                    sem_g[par]).wait()
                pltpu.make_async_copy(
                    loc_hbm.at[pl.ds(0, 128), :], rows_t.at[par, l],
                    sem_g[par]).wait()

        def compute(par):
            @plsc.parallel_loop(0, 128, unroll=4)
            def _(tok):
                tk = jnp.full((16,), tok, jnp.int32)
                vs = []
                for p in range(2):
                    sl = pl.ds(p * 16, 16)
                    a = ws[0] * rows_x[par, 0, tok, sl]
                    for l in range(1, _L):
                        a = a + ws[l] * rows_x[par, l, tok, sl]
                    vs.append(a)
                for p in range(2):
                    sl = pl.ds(p * 16, 16)
                    b = ws[_L] * rows_t[par, 0, tok, sl]
                    for l in range(1, _L):
                        b = b + ws[_L + l] * rows_t[par, l, tok, sl]
                    vs.append(b)
                for q in range(4):
                    plsc.store_scatter(out_v.at[par], [dts[q], drs[q], tk],
                                       vs[q])

        def issue_out(par, g):
            s, bc = chunk_slices(g)
            for dt in range(8):
                pltpu.async_copy(out_v.at[par, dt, :, pl.ds(0, 128)],
                                 out_hbm.at[s, dt, bc], sem_o[par])

        def drain_out(par, g):
            s, bc = chunk_slices(g)
            for dt in range(8):
                pltpu.make_async_copy(
                    out_v.at[par, dt, :, pl.ds(0, 128)],
                    out_hbm.at[s, dt, bc], sem_o[par]).wait()

        stage_idx(0, g0)
        stage_idx(1, g0 + 1)
        wait_idx(0)
        add_offsets(0)
        issue_gathers(0)

        def super_body(i, carry):
            for par in range(2):
                g = g0 + 2 * i + par
                nxt = 1 - par
                wait_gathers(par)

                @pl.when(i < _NCHW // 2 - 1)
                def _prefetch_idx():
                    stage_idx(par, g + 2)

                def _launch_next():
                    wait_idx(nxt)
                    add_offsets(nxt)
                    issue_gathers(nxt)

                if par == 0:
                    _launch_next()
                else:
                    pl.when(i < _NCHW // 2 - 1)(_launch_next)

                @pl.when(i > 0)
                def _drain_out():
                    drain_out(par, g - 2)

                compute(par)
                issue_out(par, g)
            return carry

        lax.fori_loop(0, _NCHW // 2, super_body, 0)

        last = g0 + _NCHW - 2
        drain_out(0, last)
        drain_out(1, last + 1)

    return k


_k = _make_kernel()


def kernel(x, t, loc_tables, time_tables, x_weights, t_weights):
    xp = x.astype(jnp.int32).transpose(1, 2, 0).reshape(_S, _L, _BT, 128)
    xp = xp.transpose(0, 2, 1, 3)
    tp = t.astype(jnp.int32).transpose(1, 2, 0).reshape(_S, _L, _BT, 128)
    tp = tp.transpose(0, 2, 1, 3)
    lt = lax.optimization_barrier(loc_tables.reshape(_VL, _L * _D))
    loc_flat = lt.reshape(_L * _VL, _D)
    w_all = jnp.broadcast_to(
        jnp.concatenate([x_weights, t_weights])[:, None], (2 * _L, 16))
    op = _k(xp, tp, loc_flat, time_tables, w_all)
    out = op.transpose(2, 4, 0, 1, 3).reshape(_B, _S, 2 * _D)
    return out
